# Initial kernel scaffold; baseline (speedup 1.0000x reference)
import jax
import jax.numpy as jnp
from jax.experimental import pallas as pl


def kernel(x, edge_attr, edge_index, fwd_c1_w, fwd_c1_b, fwd_c2_w, fwd_c2_b, fwd_c3_w, fwd_c3_b, fwd_lin_w, fwd_lin_b, fwd_nd_w, fwd_nd_b, bwd_c1_w, bwd_c1_b, bwd_c2_w, bwd_c2_b, bwd_c3_w, bwd_c3_b, bwd_lin_w, bwd_lin_b, bwd_nd_w, bwd_nd_b, g_c1_w, g_c1_b, g_c2_w, g_c2_b, g_c3_w, g_c3_b, g_lin_w, g_lin_b, gm_w, gm_b, fm_w, fm_b):
    raise NotImplementedError("write your pallas kernel here")


# speedup vs baseline: 88.2409x; 88.2409x over previous
import functools

import jax
import jax.numpy as jnp
from jax import lax
from jax.experimental import pallas as pl
from jax.experimental.pallas import tpu as pltpu
from jax.experimental.pallas import tpu_sc as plsc

N = 10000
E = 80000
P = 2
B = 16
HID = 64
GDIM = 50

NPAD = 10240
EPAD = 81920
TILES = 16
KSUB = EPAD // TILES // 128
ROWS_PT = NPAD // TILES
BLK = 1024


def _conv(x, w, b, stride=(1, 1), padding=((0, 0), (0, 0))):
    y = lax.conv_general_dilated(x, w, window_strides=stride, padding=padding,
                                 dimension_numbers=('NCHW', 'OIHW', 'NCHW'))
    return y + b[None, :, None, None]


def _stack(t, c1w, c1b, c2w, c2b, c3w, c3b, lw, lb):
    h = _conv(t, c1w, c1b, padding=((P, P), (0, 0)))
    h = _conv(h, c2w, c2b)
    h = _conv(h, c3w, c3b, stride=(1, B // 16))
    return h.reshape(h.shape[0], -1) @ lw.T + lb


def _affine_of_stack(cin, params):
    dim = cin * P * B
    basis = jnp.concatenate(
        [jnp.eye(dim, dtype=jnp.float32), jnp.zeros((1, dim), jnp.float32)], 0
    ).reshape(dim + 1, cin, P, B)
    out = _stack(basis, *params)
    return out[:dim] - out[dim][None], out[dim]


def _sc_body(x32, idxj, idxi, seb_hbm, z32, z16, of_x, of_s, ob_x, ob_s,
             jv, iv, seb, rows, acc_x, acc_s, sem):
    c = lax.axis_index("c")
    s = lax.axis_index("s")
    r0 = s * ROWS_PT
    pltpu.sync_copy(z32, acc_x.at[pl.ds(r0, ROWS_PT)])
    pltpu.sync_copy(z16, acc_s.at[pl.ds(r0, ROWS_PT)])
    pltpu.sync_copy(idxj.at[c, s], jv)
    pltpu.sync_copy(idxi.at[c, s], iv)
    pltpu.sync_copy(seb_hbm.at[s], seb)
    plsc.subcore_barrier()

    def step(k, carry):
        pltpu.async_copy(x32.at[jv.at[k]], rows, sem).wait()
        pltpu.sync_copy(rows, acc_x.at[iv.at[k]], add=True)
        pltpu.sync_copy(seb.at[pl.ds(k * 128, 128)], acc_s.at[iv.at[k]], add=True)
        return carry

    lax.fori_loop(0, KSUB, step, 0)
    plsc.subcore_barrier()

    @pl.when(c == 0)
    def _():
        pltpu.sync_copy(acc_x.at[pl.ds(r0, ROWS_PT)], of_x.at[pl.ds(r0, ROWS_PT)])
        pltpu.sync_copy(acc_s.at[pl.ds(r0, ROWS_PT)], of_s.at[pl.ds(r0, ROWS_PT)])

    @pl.when(c == 1)
    def _():
        pltpu.sync_copy(acc_x.at[pl.ds(r0, ROWS_PT)], ob_x.at[pl.ds(r0, ROWS_PT)])
        pltpu.sync_copy(acc_s.at[pl.ds(r0, ROWS_PT)], ob_s.at[pl.ds(r0, ROWS_PT)])


@functools.lru_cache(maxsize=1)
def _sc_segment():
    return pl.kernel(
        _sc_body,
        out_type=[
            jax.ShapeDtypeStruct((NPAD, 32), jnp.float32),
            jax.ShapeDtypeStruct((NPAD, 16), jnp.float32),
            jax.ShapeDtypeStruct((NPAD, 32), jnp.float32),
            jax.ShapeDtypeStruct((NPAD, 16), jnp.float32),
        ],
        mesh=plsc.VectorSubcoreMesh(core_axis_name="c", subcore_axis_name="s"),
        compiler_params=pltpu.CompilerParams(use_tc_tiling_on_sc=False),
        scratch_types=[
            pltpu.VMEM((KSUB, 128), jnp.int32),
            pltpu.VMEM((KSUB, 128), jnp.int32),
            pltpu.VMEM((KSUB * 128, 16), jnp.float32),
            pltpu.VMEM((128, 32), jnp.float32),
            pltpu.VMEM_SHARED((NPAD, 32), jnp.float32),
            pltpu.VMEM_SHARED((NPAD, 16), jnp.float32),
            pltpu.SemaphoreType.DMA,
        ],
    )


def _tc_body(xs, fx, fs, bx, bs, W, bg, GM, gmb, FM, fmb, out_ref):
    pid = pl.program_id(0)
    x = xs[...]
    fsv = fs[...]
    bsv = bs[...]
    degf = fsv[:, 1:2]
    degb = bsv[:, 1:2]
    U = jnp.concatenate([x * degf, x * degb, fx[...], bx[...], fsv, bsv], axis=1)
    h = jnp.dot(U, W[...], preferred_element_type=jnp.float32) + bg[...]
    g = jax.nn.sigmoid(jnp.dot(h, GM[...], preferred_element_type=jnp.float32) + gmb[...])
    hv = jnp.dot(h, FM[...], preferred_element_type=jnp.float32) + fmb[...]
    rid = pid * BLK + lax.broadcasted_iota(jnp.int32, (BLK, 1), 0)
    part = jnp.sum(jnp.where(rid < N, g * hv, 0.0), axis=0, keepdims=True)

    @pl.when(pid == 0)
    def _():
        out_ref[...] = jnp.zeros_like(out_ref)

    out_ref[...] += part


def _tc_reduce(xs, fx, fs, bx, bs, W, bg, GM, gmb, FM, fmb):
    full = lambda shape: pl.BlockSpec(shape, lambda i: (0, 0))
    return pl.pallas_call(
        _tc_body,
        grid=(NPAD // BLK,),
        in_specs=[
            pl.BlockSpec((BLK, 32), lambda i: (i, 0)),
            pl.BlockSpec((BLK, 32), lambda i: (i, 0)),
            pl.BlockSpec((BLK, 16), lambda i: (i, 0)),
            pl.BlockSpec((BLK, 32), lambda i: (i, 0)),
            pl.BlockSpec((BLK, 16), lambda i: (i, 0)),
            full((160, 64)),
            full((1, 64)),
            full((64, 128)),
            full((1, 128)),
            full((64, 128)),
            full((1, 128)),
        ],
        out_specs=pl.BlockSpec((1, 128), lambda i: (0, 0)),
        out_shape=jax.ShapeDtypeStruct((1, 128), jnp.float32),
    )(xs, fx, fs, bx, bs, W, bg, GM, gmb, FM, fmb)


def _direction_mats(stack_params, nd_w, nd_b, Ag):
    A, b = _affine_of_stack(2, stack_params)
    Wnd = nd_w.reshape(P * B, HID + 1).T
    M = A @ Wnd[:HID]
    w_ea = Wnd[HID]
    cst = b @ Wnd[:HID] + nd_b.reshape(P * B)
    return M[:32] @ Ag, M[32:] @ Ag, w_ea @ Ag, cst @ Ag


def kernel(x, edge_attr, edge_index,
           fwd_c1_w, fwd_c1_b, fwd_c2_w, fwd_c2_b, fwd_c3_w, fwd_c3_b,
           fwd_lin_w, fwd_lin_b, fwd_nd_w, fwd_nd_b,
           bwd_c1_w, bwd_c1_b, bwd_c2_w, bwd_c2_b, bwd_c3_w, bwd_c3_b,
           bwd_lin_w, bwd_lin_b, bwd_nd_w, bwd_nd_b,
           g_c1_w, g_c1_b, g_c2_w, g_c2_b, g_c3_w, g_c3_b,
           g_lin_w, g_lin_b, gm_w, gm_b, fm_w, fm_b):
    f32 = jnp.float32
    Ag, bg = _affine_of_stack(1, (g_c1_w, g_c1_b, g_c2_w, g_c2_b,
                                  g_c3_w, g_c3_b, g_lin_w, g_lin_b))
    Mfi, Mfj, wf, cf = _direction_mats(
        (fwd_c1_w, fwd_c1_b, fwd_c2_w, fwd_c2_b, fwd_c3_w, fwd_c3_b,
         fwd_lin_w, fwd_lin_b), fwd_nd_w, fwd_nd_b, Ag)
    Mbi, Mbj, wb, cb = _direction_mats(
        (bwd_c1_w, bwd_c1_b, bwd_c2_w, bwd_c2_b, bwd_c3_w, bwd_c3_b,
         bwd_lin_w, bwd_lin_b), bwd_nd_w, bwd_nd_b, Ag)
    Afs = jnp.zeros((16, HID), f32).at[0].set(wf).at[1].set(cf)
    Abs = jnp.zeros((16, HID), f32).at[0].set(wb).at[1].set(cb)
    W_all = jnp.concatenate([Mfi, Mbi, Mfj, Mbj, Afs, Abs], axis=0)
    GM = jnp.zeros((HID, 128), f32).at[:, :GDIM].set(gm_w.T)
    gmb = jnp.zeros((1, 128), f32).at[0, :GDIM].set(gm_b)
    FM = jnp.zeros((HID, 128), f32).at[:, :GDIM].set(fm_w.T)
    fmb = jnp.zeros((1, 128), f32).at[0, :GDIM].set(fm_b)

    x32p = jnp.zeros((NPAD, 32), f32).at[:N].set(x.reshape(N, P * B))
    pad_i = jnp.full((EPAD - E,), N, jnp.int32)
    srcp = jnp.concatenate([edge_index[0], pad_i])
    dstp = jnp.concatenate([edge_index[1], pad_i])
    idxj = jnp.stack([srcp, dstp]).reshape(2, TILES, KSUB, 128)
    idxi = jnp.stack([dstp, srcp]).reshape(2, TILES, KSUB, 128)
    eap = jnp.concatenate([edge_attr, jnp.zeros((EPAD - E,), f32)])
    seb = (jnp.zeros((EPAD, 16), f32).at[:, 0].set(eap).at[:, 1].set(1.0)
           ).reshape(TILES, KSUB * 128, 16)
    z32 = jnp.zeros((ROWS_PT, 32), f32)
    z16 = jnp.zeros((ROWS_PT, 16), f32)

    fx, fs, bx, bs = _sc_segment()(x32p, idxj, idxi, seb, z32, z16)

    out = _tc_reduce(x32p, fx, fs, bx, bs, W_all,
                     bg.reshape(1, HID), GM, gmb, FM, fmb)
    return out[0, :GDIM]


# speedup vs baseline: 94.2641x; 1.0683x over previous
import functools

import jax
import jax.numpy as jnp
from jax import lax
from jax.experimental import pallas as pl
from jax.experimental.pallas import tpu as pltpu
from jax.experimental.pallas import tpu_sc as plsc

N = 10000
E = 80000
P = 2
B = 16
HID = 64
GDIM = 50

NPAD = 10240
EPAD = 81920
TILES = 16
KSUB = EPAD // TILES // 128
ROWS_PT = NPAD // TILES
BLK = 1024


def _conv(x, w, b, stride=(1, 1), padding=((0, 0), (0, 0))):
    y = lax.conv_general_dilated(x, w, window_strides=stride, padding=padding,
                                 dimension_numbers=('NCHW', 'OIHW', 'NCHW'))
    return y + b[None, :, None, None]


def _stack(t, c1w, c1b, c2w, c2b, c3w, c3b, lw, lb):
    h = _conv(t, c1w, c1b, padding=((P, P), (0, 0)))
    h = _conv(h, c2w, c2b)
    h = _conv(h, c3w, c3b, stride=(1, B // 16))
    return h.reshape(h.shape[0], -1) @ lw.T + lb


def _affine_of_stack(cin, params):
    dim = cin * P * B
    basis = jnp.concatenate(
        [jnp.eye(dim, dtype=jnp.float32), jnp.zeros((1, dim), jnp.float32)], 0
    ).reshape(dim + 1, cin, P, B)
    out = _stack(basis, *params)
    return out[:dim] - out[dim][None], out[dim]


def _sc_body(x32, idxj, idxi, seb_hbm, z32, z16, of_x, of_s, ob_x, ob_s,
             jv, iv, seb, rows, acc_x, acc_s, sem):
    c = lax.axis_index("c")
    s = lax.axis_index("s")
    r0 = s * ROWS_PT
    pltpu.sync_copy(z32, acc_x.at[pl.ds(r0, ROWS_PT)])
    pltpu.sync_copy(z16, acc_s.at[pl.ds(r0, ROWS_PT)])
    pltpu.sync_copy(idxj.at[c, s], jv)
    pltpu.sync_copy(idxi.at[c, s], iv)
    pltpu.sync_copy(seb_hbm.at[s], seb)
    plsc.subcore_barrier()

    for b in range(2):
        pltpu.async_copy(x32.at[jv.at[b]], rows.at[b], sem)

    def step(g, carry):
        for b in range(2):
            k = 2 * g + b
            pltpu.make_async_copy(x32.at[jv.at[k]], rows.at[b], sem).wait()
            pltpu.sync_copy(rows.at[b], acc_x.at[iv.at[k]], add=True)
            pltpu.sync_copy(seb.at[pl.ds(k * 128, 128)], acc_s.at[iv.at[k]],
                            add=True)

            @pl.when(k + 2 < KSUB)
            def _():
                pltpu.async_copy(x32.at[jv.at[k + 2]], rows.at[b], sem)
        return carry

    lax.fori_loop(0, KSUB // 2, step, 0)
    plsc.subcore_barrier()

    @pl.when(c == 0)
    def _():
        pltpu.sync_copy(acc_x.at[pl.ds(r0, ROWS_PT)], of_x.at[pl.ds(r0, ROWS_PT)])
        pltpu.sync_copy(acc_s.at[pl.ds(r0, ROWS_PT)], of_s.at[pl.ds(r0, ROWS_PT)])

    @pl.when(c == 1)
    def _():
        pltpu.sync_copy(acc_x.at[pl.ds(r0, ROWS_PT)], ob_x.at[pl.ds(r0, ROWS_PT)])
        pltpu.sync_copy(acc_s.at[pl.ds(r0, ROWS_PT)], ob_s.at[pl.ds(r0, ROWS_PT)])


@functools.lru_cache(maxsize=1)
def _sc_segment():
    return pl.kernel(
        _sc_body,
        out_type=[
            jax.ShapeDtypeStruct((NPAD, 32), jnp.float32),
            jax.ShapeDtypeStruct((NPAD, 16), jnp.float32),
            jax.ShapeDtypeStruct((NPAD, 32), jnp.float32),
            jax.ShapeDtypeStruct((NPAD, 16), jnp.float32),
        ],
        mesh=plsc.VectorSubcoreMesh(core_axis_name="c", subcore_axis_name="s"),
        compiler_params=pltpu.CompilerParams(use_tc_tiling_on_sc=False),
        scratch_types=[
            pltpu.VMEM((KSUB, 128), jnp.int32),
            pltpu.VMEM((KSUB, 128), jnp.int32),
            pltpu.VMEM((KSUB * 128, 16), jnp.float32),
            pltpu.VMEM((2, 128, 32), jnp.float32),
            pltpu.VMEM_SHARED((NPAD, 32), jnp.float32),
            pltpu.VMEM_SHARED((NPAD, 16), jnp.float32),
            pltpu.SemaphoreType.DMA,
        ],
    )


def _tc_body(xs, fx, fs, bx, bs, W, bg, GM, gmb, FM, fmb, out_ref):
    pid = pl.program_id(0)
    x = xs[...]
    fsv = fs[...]
    bsv = bs[...]
    degf = fsv[:, 1:2]
    degb = bsv[:, 1:2]
    U = jnp.concatenate([x * degf, x * degb, fx[...], bx[...], fsv, bsv], axis=1)
    h = jnp.dot(U, W[...], preferred_element_type=jnp.float32) + bg[...]
    g = jax.nn.sigmoid(jnp.dot(h, GM[...], preferred_element_type=jnp.float32) + gmb[...])
    hv = jnp.dot(h, FM[...], preferred_element_type=jnp.float32) + fmb[...]
    rid = pid * BLK + lax.broadcasted_iota(jnp.int32, (BLK, 1), 0)
    part = jnp.sum(jnp.where(rid < N, g * hv, 0.0), axis=0, keepdims=True)

    @pl.when(pid == 0)
    def _():
        out_ref[...] = jnp.zeros_like(out_ref)

    out_ref[...] += part


def _tc_reduce(xs, fx, fs, bx, bs, W, bg, GM, gmb, FM, fmb):
    full = lambda shape: pl.BlockSpec(shape, lambda i: (0, 0))
    return pl.pallas_call(
        _tc_body,
        grid=(NPAD // BLK,),
        in_specs=[
            pl.BlockSpec((BLK, 32), lambda i: (i, 0)),
            pl.BlockSpec((BLK, 32), lambda i: (i, 0)),
            pl.BlockSpec((BLK, 16), lambda i: (i, 0)),
            pl.BlockSpec((BLK, 32), lambda i: (i, 0)),
            pl.BlockSpec((BLK, 16), lambda i: (i, 0)),
            full((160, 64)),
            full((1, 64)),
            full((64, 128)),
            full((1, 128)),
            full((64, 128)),
            full((1, 128)),
        ],
        out_specs=pl.BlockSpec((1, 128), lambda i: (0, 0)),
        out_shape=jax.ShapeDtypeStruct((1, 128), jnp.float32),
    )(xs, fx, fs, bx, bs, W, bg, GM, gmb, FM, fmb)


def _direction_mats(stack_params, nd_w, nd_b, Ag):
    A, b = _affine_of_stack(2, stack_params)
    Wnd = nd_w.reshape(P * B, HID + 1).T
    M = A @ Wnd[:HID]
    w_ea = Wnd[HID]
    cst = b @ Wnd[:HID] + nd_b.reshape(P * B)
    return M[:32] @ Ag, M[32:] @ Ag, w_ea @ Ag, cst @ Ag


def kernel(x, edge_attr, edge_index,
           fwd_c1_w, fwd_c1_b, fwd_c2_w, fwd_c2_b, fwd_c3_w, fwd_c3_b,
           fwd_lin_w, fwd_lin_b, fwd_nd_w, fwd_nd_b,
           bwd_c1_w, bwd_c1_b, bwd_c2_w, bwd_c2_b, bwd_c3_w, bwd_c3_b,
           bwd_lin_w, bwd_lin_b, bwd_nd_w, bwd_nd_b,
           g_c1_w, g_c1_b, g_c2_w, g_c2_b, g_c3_w, g_c3_b,
           g_lin_w, g_lin_b, gm_w, gm_b, fm_w, fm_b):
    f32 = jnp.float32
    Ag, bg = _affine_of_stack(1, (g_c1_w, g_c1_b, g_c2_w, g_c2_b,
                                  g_c3_w, g_c3_b, g_lin_w, g_lin_b))
    Mfi, Mfj, wf, cf = _direction_mats(
        (fwd_c1_w, fwd_c1_b, fwd_c2_w, fwd_c2_b, fwd_c3_w, fwd_c3_b,
         fwd_lin_w, fwd_lin_b), fwd_nd_w, fwd_nd_b, Ag)
    Mbi, Mbj, wb, cb = _direction_mats(
        (bwd_c1_w, bwd_c1_b, bwd_c2_w, bwd_c2_b, bwd_c3_w, bwd_c3_b,
         bwd_lin_w, bwd_lin_b), bwd_nd_w, bwd_nd_b, Ag)
    Afs = jnp.zeros((16, HID), f32).at[0].set(wf).at[1].set(cf)
    Abs = jnp.zeros((16, HID), f32).at[0].set(wb).at[1].set(cb)
    W_all = jnp.concatenate([Mfi, Mbi, Mfj, Mbj, Afs, Abs], axis=0)
    GM = jnp.zeros((HID, 128), f32).at[:, :GDIM].set(gm_w.T)
    gmb = jnp.zeros((1, 128), f32).at[0, :GDIM].set(gm_b)
    FM = jnp.zeros((HID, 128), f32).at[:, :GDIM].set(fm_w.T)
    fmb = jnp.zeros((1, 128), f32).at[0, :GDIM].set(fm_b)

    x32p = jnp.zeros((NPAD, 32), f32).at[:N].set(x.reshape(N, P * B))
    pad_i = jnp.full((EPAD - E,), N, jnp.int32)
    srcp = jnp.concatenate([edge_index[0], pad_i])
    dstp = jnp.concatenate([edge_index[1], pad_i])
    idxj = jnp.stack([srcp, dstp]).reshape(2, TILES, KSUB, 128)
    idxi = jnp.stack([dstp, srcp]).reshape(2, TILES, KSUB, 128)
    eap = jnp.concatenate([edge_attr, jnp.zeros((EPAD - E,), f32)])
    seb = (jnp.zeros((EPAD, 16), f32).at[:, 0].set(eap).at[:, 1].set(1.0)
           ).reshape(TILES, KSUB * 128, 16)
    z32 = jnp.zeros((ROWS_PT, 32), f32)
    z16 = jnp.zeros((ROWS_PT, 16), f32)

    fx, fs, bx, bs = _sc_segment()(x32p, idxj, idxi, seb, z32, z16)

    out = _tc_reduce(x32p, fx, fs, bx, bs, W_all,
                     bg.reshape(1, HID), GM, gmb, FM, fmb)
    return out[0, :GDIM]
